# Initial kernel scaffold; baseline (speedup 1.0000x reference)
import jax
import jax.numpy as jnp
from jax.experimental import pallas as pl
from jax.experimental.pallas import tpu as pltpu

B, T, FX = 256, 64, 128
INFO = 256
EM_IN = 128
OH_OUT = 64
N_LAYER = 1024
NUM_LAYERS = 4
TOTAL = N_LAYER * NUM_LAYERS
K = 512


def _mask_from_logits(z, out_ref, li):
    zi = jax.lax.bitcast_convert_type(z, jnp.int32)
    key = zi ^ (jax.lax.shift_right_arithmetic(zi, 31) & jnp.int32(0x7FFFFFFF))

    thr0 = jnp.full((B, 1), jnp.int32(-2147483648))
    bit0 = jnp.int32(1 << 30)

    def bstep(j, carry):
        thr, bit = carry
        t = thr + bit
        cnt = jnp.sum((key >= t).astype(jnp.int32), axis=1, keepdims=True)
        thr = jnp.where(cnt >= K, t, thr)
        return thr, jax.lax.shift_right_logical(bit, 1)

    thr, _ = jax.lax.fori_loop(0, 31, bstep, (thr0, bit0))

    gt = key > thr
    eq = key == thr
    cgt = jnp.sum(gt.astype(jnp.int32), axis=1, keepdims=True)
    need = K - cgt
    pos = jax.lax.broadcasted_iota(jnp.int32, (B, N_LAYER), 1)

    def tstep(j, carry):
        m, bit = carry
        cand = m + bit
        cnt = jnp.sum((eq & (pos < cand)).astype(jnp.int32), axis=1, keepdims=True)
        m = jnp.where(cnt <= need, cand, m)
        return m, jax.lax.shift_right_logical(bit, 1)

    m, _ = jax.lax.fori_loop(0, 11, tstep,
                             (jnp.zeros((B, 1), jnp.int32), jnp.int32(1 << 10)))

    mask = gt | (eq & (pos < m))
    out_ref[li] = mask.astype(jnp.float32)


def _body(xT_ref, e_ref, Wih_ref, Whh_ref, b_ref,
          m1_ref, mb1_ref, m2_ref, mb2_ref,
          g1e_ref, g1t_ref, gb1_ref, g2_ref, gb2_ref, g3_ref, gb3_ref,
          G_ref, out_ref, h_ref, c_ref):
    h_ref[...] = jnp.zeros((B, INFO), jnp.float32)
    c_ref[...] = jnp.zeros((B, INFO), jnp.float32)

    def step(t, carry):
        xt = xT_ref[t]
        gates = (jnp.dot(xt, Wih_ref[...], preferred_element_type=jnp.float32)
                 + jnp.dot(h_ref[...], Whh_ref[...], preferred_element_type=jnp.float32)
                 + b_ref[...])
        i = jax.nn.sigmoid(gates[:, :INFO])
        f = jax.nn.sigmoid(gates[:, INFO:2 * INFO])
        g = jnp.tanh(gates[:, 2 * INFO:3 * INFO])
        o = jax.nn.sigmoid(gates[:, 3 * INFO:])
        c = f * c_ref[...] + i * g
        c_ref[...] = c
        h_ref[...] = o * jnp.tanh(c)
        return carry

    jax.lax.fori_loop(0, T, step, 0)
    traj = h_ref[...]

    emb = (jnp.dot(
        jax.nn.relu(jnp.dot(e_ref[...], m1_ref[...],
                            preferred_element_type=jnp.float32) + mb1_ref[...]),
        m2_ref[...], preferred_element_type=jnp.float32) + mb2_ref[...])

    h1 = jax.nn.relu(
        jnp.dot(emb, g1e_ref[...], preferred_element_type=jnp.float32)
        + jnp.dot(traj, g1t_ref[...], preferred_element_type=jnp.float32)
        + gb1_ref[...])
    h2 = jax.nn.relu(
        jnp.dot(h1, g2_ref[...], preferred_element_type=jnp.float32) + gb2_ref[...])
    mv = jnp.dot(h2, g3_ref[...], preferred_element_type=jnp.float32) + gb3_ref[...]

    for li in range(NUM_LAYERS):
        z = mv[:, li * N_LAYER:(li + 1) * N_LAYER] + G_ref[li]
        _mask_from_logits(z, out_ref, li)


def kernel(x, embedding_input, W_ih, W_hh, b_ih, b_hh,
           mlp_w1, mlp_b1, mlp_w2, mlp_b2,
           g_w1, g_b1, g_w2, g_b2, g_w3, g_b3):
    xT = jnp.swapaxes(x, 0, 1)
    e = jnp.squeeze(embedding_input, axis=1)
    b = (b_ih + b_hh).reshape(1, 4 * INFO)
    g1e = g_w1[:OH_OUT]
    g1t = g_w1[OH_OUT:]

    gkey = jax.random.key(42)
    G = jnp.stack([
        jax.random.gumbel(jax.random.fold_in(gkey, li), (B, N_LAYER), jnp.float32)
        for li in range(NUM_LAYERS)
    ], axis=0)

    return pl.pallas_call(
        _body,
        out_shape=jax.ShapeDtypeStruct((NUM_LAYERS, B, N_LAYER), jnp.float32),
        scratch_shapes=[
            pltpu.VMEM((B, INFO), jnp.float32),
            pltpu.VMEM((B, INFO), jnp.float32),
        ],
    )(xT, e, W_ih, W_hh, b,
      mlp_w1, mlp_b1.reshape(1, -1), mlp_w2, mlp_b2.reshape(1, -1),
      g1e, g1t, g_b1.reshape(1, -1), g_w2, g_b2.reshape(1, -1), g_w3,
      g_b3.reshape(1, -1), G)


# speedup vs baseline: 15.6689x; 15.6689x over previous
import jax
import jax.numpy as jnp
from jax.experimental import pallas as pl
from jax.experimental.pallas import tpu as pltpu

B, T, FX = 256, 64, 128
INFO = 256
EM_IN = 128
OH_OUT = 64
N_LAYER = 1024
NUM_LAYERS = 4
TOTAL = N_LAYER * NUM_LAYERS
K = 512


def _mask_from_logits(z, out_ref, li):
    zi = jax.lax.bitcast_convert_type(z, jnp.int32)
    key = zi ^ (jax.lax.shift_right_arithmetic(zi, 31) & jnp.int32(0x7FFFFFFF))

    cnt0 = jnp.sum((key >= 0).astype(jnp.int32), axis=1, keepdims=True)
    thr0 = jnp.where(cnt0 >= K, jnp.int32(0), jnp.int32(-2147483648))
    bit0 = jnp.int32(1 << 30)

    def bstep(j, carry):
        thr, bit = carry
        t = thr + bit
        cnt = jnp.sum((key >= t).astype(jnp.int32), axis=1, keepdims=True)
        thr = jnp.where(cnt >= K, t, thr)
        return thr, jax.lax.shift_right_logical(bit, 1)

    thr, _ = jax.lax.fori_loop(0, 31, bstep, (thr0, bit0))

    gt = key > thr
    eq = key == thr
    cgt = jnp.sum(gt.astype(jnp.int32), axis=1, keepdims=True)
    need = K - cgt
    pos = jax.lax.broadcasted_iota(jnp.int32, (B, N_LAYER), 1)

    def tstep(j, carry):
        m, bit = carry
        cand = m + bit
        cnt = jnp.sum((eq & (pos < cand)).astype(jnp.int32), axis=1, keepdims=True)
        m = jnp.where(cnt <= need, cand, m)
        return m, jax.lax.shift_right_logical(bit, 1)

    m, _ = jax.lax.fori_loop(0, 11, tstep,
                             (jnp.zeros((B, 1), jnp.int32), jnp.int32(1 << 10)))

    mask = gt | (eq & (pos < m))
    out_ref[li] = mask.astype(jnp.float32)


def _body(xT_ref, e_ref, Wih_ref, Whh_ref, b_ref,
          m1_ref, mb1_ref, m2_ref, mb2_ref,
          g1e_ref, g1t_ref, gb1_ref, g2_ref, gb2_ref, g3_ref, gb3_ref,
          G_ref, out_ref, h_ref, c_ref):
    h_ref[...] = jnp.zeros((B, INFO), jnp.float32)
    c_ref[...] = jnp.zeros((B, INFO), jnp.float32)

    def step(t, carry):
        xt = xT_ref[t]
        gates = (jnp.dot(xt, Wih_ref[...], preferred_element_type=jnp.float32)
                 + jnp.dot(h_ref[...], Whh_ref[...], preferred_element_type=jnp.float32)
                 + b_ref[...])
        i = jax.nn.sigmoid(gates[:, :INFO])
        f = jax.nn.sigmoid(gates[:, INFO:2 * INFO])
        g = jnp.tanh(gates[:, 2 * INFO:3 * INFO])
        o = jax.nn.sigmoid(gates[:, 3 * INFO:])
        c = f * c_ref[...] + i * g
        c_ref[...] = c
        h_ref[...] = o * jnp.tanh(c)
        return carry

    jax.lax.fori_loop(0, T, step, 0)
    traj = h_ref[...]

    emb = (jnp.dot(
        jax.nn.relu(jnp.dot(e_ref[...], m1_ref[...],
                            preferred_element_type=jnp.float32) + mb1_ref[...]),
        m2_ref[...], preferred_element_type=jnp.float32) + mb2_ref[...])

    h1 = jax.nn.relu(
        jnp.dot(emb, g1e_ref[...], preferred_element_type=jnp.float32)
        + jnp.dot(traj, g1t_ref[...], preferred_element_type=jnp.float32)
        + gb1_ref[...])
    h2 = jax.nn.relu(
        jnp.dot(h1, g2_ref[...], preferred_element_type=jnp.float32) + gb2_ref[...])
    mv = jnp.dot(h2, g3_ref[...], preferred_element_type=jnp.float32) + gb3_ref[...]

    for li in range(NUM_LAYERS):
        z = mv[:, li * N_LAYER:(li + 1) * N_LAYER] + G_ref[li]
        _mask_from_logits(z, out_ref, li)


def kernel(x, embedding_input, W_ih, W_hh, b_ih, b_hh,
           mlp_w1, mlp_b1, mlp_w2, mlp_b2,
           g_w1, g_b1, g_w2, g_b2, g_w3, g_b3):
    xT = jnp.swapaxes(x, 0, 1)
    e = jnp.squeeze(embedding_input, axis=1)
    b = (b_ih + b_hh).reshape(1, 4 * INFO)
    g1e = g_w1[:OH_OUT]
    g1t = g_w1[OH_OUT:]

    gkey = jax.random.key(42)
    G = jnp.stack([
        jax.random.gumbel(jax.random.fold_in(gkey, li), (B, N_LAYER), jnp.float32)
        for li in range(NUM_LAYERS)
    ], axis=0)

    return pl.pallas_call(
        _body,
        out_shape=jax.ShapeDtypeStruct((NUM_LAYERS, B, N_LAYER), jnp.float32),
        scratch_shapes=[
            pltpu.VMEM((B, INFO), jnp.float32),
            pltpu.VMEM((B, INFO), jnp.float32),
        ],
    )(xT, e, W_ih, W_hh, b,
      mlp_w1, mlp_b1.reshape(1, -1), mlp_w2, mlp_b2.reshape(1, -1),
      g1e, g1t, g_b1.reshape(1, -1), g_w2, g_b2.reshape(1, -1), g_w3,
      g_b3.reshape(1, -1), G)
